# baseline (speedup 1.0000x reference)
import jax
import jax.numpy as jnp
from jax.experimental import pallas as pl
from jax.experimental.pallas import tpu as pltpu

CODEBOOK_SIZE = 8192
TOKEN_SIZE = 256
COMMITMENT_COST = 0.25

N_TOKENS = 16384
ROW_BLK = 256
N_STEPS = N_TOKENS // ROW_BLK


def _vq_body(z_ref, e_ref, idx_ref, zq_ref, loss_ref, com_ref, cod_ref):
    z = z_ref[...]
    e = e_ref[...]
    zn = jnp.sum(z * z, axis=1, keepdims=True)
    en = jnp.sum(e * e, axis=1)
    mm = jax.lax.dot_general(
        z, e, (((1,), (1,)), ((), ())),
        preferred_element_type=jnp.float32)
    d = (zn + en) - 2.0 * mm
    dmin = jnp.min(d, axis=1, keepdims=True)
    iota = jax.lax.broadcasted_iota(jnp.int32, d.shape, 1)
    idx = jnp.min(jnp.where(d == dmin, iota, CODEBOOK_SIZE), axis=1)
    idx = idx.astype(jnp.int32)
    idx_ref[0, 0, :] = idx

    onehot = (iota == idx[:, None]).astype(jnp.float32)
    zq = jax.lax.dot_general(
        onehot, e, (((1,), (0,)), ((), ())),
        preferred_element_type=jnp.float32,
        precision=jax.lax.Precision.HIGHEST)
    diff = zq - z
    d2 = diff * diff
    com = COMMITMENT_COST * d2
    zq_ref[...] = z + diff
    loss_ref[...] = com + d2
    com_ref[...] = com
    cod_ref[...] = d2


def kernel(z, embedding):
    b, c, h, w = z.shape
    z_flat = jnp.transpose(z.astype(jnp.float32), (0, 2, 3, 1)).reshape(-1, c)

    out_shapes = (
        jax.ShapeDtypeStruct((N_STEPS, 1, ROW_BLK), jnp.int32),
        jax.ShapeDtypeStruct((N_TOKENS, c), jnp.float32),
        jax.ShapeDtypeStruct((N_TOKENS, c), jnp.float32),
        jax.ShapeDtypeStruct((N_TOKENS, c), jnp.float32),
        jax.ShapeDtypeStruct((N_TOKENS, c), jnp.float32),
    )
    grid = (N_STEPS,)
    in_specs = [
        pl.BlockSpec((ROW_BLK, c), lambda i: (i, 0)),
        pl.BlockSpec((CODEBOOK_SIZE, c), lambda i: (0, 0)),
    ]
    out_specs = (
        pl.BlockSpec((1, 1, ROW_BLK), lambda i: (i, 0, 0)),
        pl.BlockSpec((ROW_BLK, c), lambda i: (i, 0)),
        pl.BlockSpec((ROW_BLK, c), lambda i: (i, 0)),
        pl.BlockSpec((ROW_BLK, c), lambda i: (i, 0)),
        pl.BlockSpec((ROW_BLK, c), lambda i: (i, 0)),
    )
    idx, zq, loss, com, cod = pl.pallas_call(
        _vq_body,
        grid=grid,
        in_specs=in_specs,
        out_specs=out_specs,
        out_shape=out_shapes,
        compiler_params=pltpu.CompilerParams(
            dimension_semantics=("arbitrary",),
        ),
    )(z_flat, embedding)

    z_quantized = zq.reshape(b, h, w, c).transpose(0, 3, 1, 2)
    loss = loss.reshape(b, h, w, c)
    com = com.reshape(b, h, w, c)
    cod = cod.reshape(b, h, w, c)
    indices = idx.reshape(-1)
    return (z_quantized, loss, com, cod, indices)


# speedup vs baseline: 2.4681x; 2.4681x over previous
import functools

import jax
import jax.numpy as jnp
from jax import lax
from jax.experimental import pallas as pl
from jax.experimental.pallas import tpu as pltpu
from jax.experimental.pallas import tpu_sc as plsc

CB = 8192
C = 256
COMMIT = 0.25
N = 16384
ROW_BLK = 256
N_STEPS = N // ROW_BLK


def _argmin_body(z_ref, e_ref, idx_ref):
    z = z_ref[...]
    e = e_ref[...]
    zn = jnp.sum(z * z, axis=1, keepdims=True)
    en = jnp.sum(e * e, axis=1)
    mm = lax.dot_general(z, e, (((1,), (1,)), ((), ())),
                         preferred_element_type=jnp.float32)
    d = (zn + en) - 2.0 * mm
    dmin = jnp.min(d, axis=1, keepdims=True)
    iota = lax.broadcasted_iota(jnp.int32, d.shape, 1)
    idx = jnp.min(jnp.where(d == dmin, iota, CB), axis=1)
    idx_ref[0, 0, :] = idx.astype(jnp.int32)


def _losses_body(z_ref, zq_ref, out_ref, loss_ref, com_ref, cod_ref):
    z = z_ref[...]
    zq = zq_ref[...]
    diff = zq - z
    d2 = diff * diff
    com = COMMIT * d2
    out_ref[...] = z + diff
    loss_ref[...] = com + d2
    com_ref[...] = com
    cod_ref[...] = d2


def _make_sc_gather():
    info = plsc.get_sparse_core_info()
    nw = info.num_cores * info.num_subcores
    b_per_w = N // nw
    chunk = 32
    nch = b_per_w // chunk
    mesh = plsc.VectorSubcoreMesh(core_axis_name="c", subcore_axis_name="s")

    @functools.partial(
        pl.kernel, mesh=mesh,
        out_type=jax.ShapeDtypeStruct((N, C), jnp.float32),
        scratch_types=[
            pltpu.VMEM((chunk,), jnp.int32),
            pltpu.VMEM((chunk, C), jnp.float32),
            pltpu.SemaphoreType.DMA,
        ],
    )
    def sc_gather(table_hbm, idx_hbm, out_hbm, idx_v, rows_v, sem):
        wid = lax.axis_index("s") * info.num_cores + lax.axis_index("c")
        base = wid * b_per_w

        def body(i, _):
            off = base + i * chunk
            pltpu.sync_copy(idx_hbm.at[pl.ds(off, chunk)], idx_v)
            pltpu.async_copy(table_hbm.at[idx_v], rows_v, sem).wait()
            pltpu.sync_copy(rows_v, out_hbm.at[pl.ds(off, chunk)])
            return ()

        lax.fori_loop(0, nch, body, (), unroll=False)

    return sc_gather


def kernel(z, embedding):
    b, c, h, w = z.shape
    z_flat = jnp.transpose(z.astype(jnp.float32), (0, 2, 3, 1)).reshape(-1, c)

    idx3 = pl.pallas_call(
        _argmin_body,
        grid=(N_STEPS,),
        in_specs=[
            pl.BlockSpec((ROW_BLK, c), lambda i: (i, 0)),
            pl.BlockSpec((CB, c), lambda i: (0, 0)),
        ],
        out_specs=pl.BlockSpec((1, 1, ROW_BLK), lambda i: (i, 0, 0)),
        out_shape=jax.ShapeDtypeStruct((N_STEPS, 1, ROW_BLK), jnp.int32),
        compiler_params=pltpu.CompilerParams(
            dimension_semantics=("arbitrary",)),
    )(z_flat, embedding)
    indices = idx3.reshape(-1)

    zq = _make_sc_gather()(embedding, indices)

    LB = 2048
    zq_out, loss, com, cod = pl.pallas_call(
        _losses_body,
        grid=(N // LB,),
        in_specs=[
            pl.BlockSpec((LB, c), lambda i: (i, 0)),
            pl.BlockSpec((LB, c), lambda i: (i, 0)),
        ],
        out_specs=tuple(pl.BlockSpec((LB, c), lambda i: (i, 0)) for _ in range(4)),
        out_shape=tuple(jax.ShapeDtypeStruct((N, c), jnp.float32) for _ in range(4)),
        compiler_params=pltpu.CompilerParams(
            dimension_semantics=("arbitrary",)),
    )(z_flat, zq)

    z_quantized = zq_out.reshape(b, h, w, c).transpose(0, 3, 1, 2)
    loss = loss.reshape(b, h, w, c)
    com = com.reshape(b, h, w, c)
    cod = cod.reshape(b, h, w, c)
    return (z_quantized, loss, com, cod, indices)


# speedup vs baseline: 2.8198x; 1.1425x over previous
import functools

import jax
import jax.numpy as jnp
from jax import lax
from jax.experimental import pallas as pl
from jax.experimental.pallas import tpu as pltpu
from jax.experimental.pallas import tpu_sc as plsc

CB = 8192
C = 256
COMMIT = 0.25
N = 16384
ROW_BLK = 512
N_STEPS = N // ROW_BLK


def _argmin_body(z_ref, e_ref, idx_ref, en_ref):
    z = z_ref[...]
    e = e_ref[...]
    zn = jnp.sum(z * z, axis=1, keepdims=True)

    @pl.when(pl.program_id(0) == 0)
    def _():
        en_ref[...] = jnp.sum(e * e, axis=1, keepdims=True)

    en = en_ref[...][:, 0]
    mm = lax.dot_general(z, e, (((1,), (1,)), ((), ())),
                         preferred_element_type=jnp.float32)
    d = (zn + en) - 2.0 * mm
    dmin = jnp.min(d, axis=1, keepdims=True)
    iota = lax.broadcasted_iota(jnp.int32, d.shape, 1)
    idx = jnp.min(jnp.where(d == dmin, iota, CB), axis=1)
    idx_ref[0, 0, :] = idx.astype(jnp.int32)


def _losses_body(z_ref, zq_ref, out_ref, loss_ref, com_ref, cod_ref):
    z = z_ref[...]
    zq = zq_ref[...]
    diff = zq - z
    d2 = diff * diff
    com = COMMIT * d2
    out_ref[...] = z + diff
    loss_ref[...] = com + d2
    com_ref[...] = com
    cod_ref[...] = d2


def _make_sc_gather():
    info = plsc.get_sparse_core_info()
    nw = info.num_cores * info.num_subcores
    b_per_w = N // nw
    chunk = 32
    nch = b_per_w // chunk
    mesh = plsc.VectorSubcoreMesh(core_axis_name="c", subcore_axis_name="s")

    @functools.partial(
        pl.kernel, mesh=mesh,
        out_type=jax.ShapeDtypeStruct((N, C), jnp.float32),
        scratch_types=[
            pltpu.VMEM((chunk,), jnp.int32),
            pltpu.VMEM((chunk, C), jnp.float32),
            pltpu.SemaphoreType.DMA,
        ],
    )
    def sc_gather(table_hbm, idx_hbm, out_hbm, idx_v, rows_v, sem):
        wid = lax.axis_index("s") * info.num_cores + lax.axis_index("c")
        base = wid * b_per_w

        def body(i, _):
            off = base + i * chunk
            pltpu.sync_copy(idx_hbm.at[pl.ds(off, chunk)], idx_v)
            pltpu.async_copy(table_hbm.at[idx_v], rows_v, sem).wait()
            pltpu.sync_copy(rows_v, out_hbm.at[pl.ds(off, chunk)])
            return ()

        lax.fori_loop(0, nch, body, (), unroll=False)

    return sc_gather


def kernel(z, embedding):
    b, c, h, w = z.shape
    z_flat = jnp.transpose(z.astype(jnp.float32), (0, 2, 3, 1)).reshape(-1, c)

    idx3 = pl.pallas_call(
        _argmin_body,
        grid=(N_STEPS,),
        in_specs=[
            pl.BlockSpec((ROW_BLK, c), lambda i: (i, 0)),
            pl.BlockSpec((CB, c), lambda i: (0, 0)),
        ],
        out_specs=pl.BlockSpec((1, 1, ROW_BLK), lambda i: (i, 0, 0)),
        out_shape=jax.ShapeDtypeStruct((N_STEPS, 1, ROW_BLK), jnp.int32),
        scratch_shapes=[pltpu.VMEM((CB, 1), jnp.float32)],
        compiler_params=pltpu.CompilerParams(
            dimension_semantics=("arbitrary",)),
    )(z_flat, embedding)
    indices = idx3.reshape(-1)

    zq = _make_sc_gather()(embedding, indices)

    LB = 2048
    zq_out, loss, com, cod = pl.pallas_call(
        _losses_body,
        grid=(N // LB,),
        in_specs=[
            pl.BlockSpec((LB, c), lambda i: (i, 0)),
            pl.BlockSpec((LB, c), lambda i: (i, 0)),
        ],
        out_specs=tuple(pl.BlockSpec((LB, c), lambda i: (i, 0)) for _ in range(4)),
        out_shape=tuple(jax.ShapeDtypeStruct((N, c), jnp.float32) for _ in range(4)),
        compiler_params=pltpu.CompilerParams(
            dimension_semantics=("arbitrary",)),
    )(z_flat, zq)

    z_quantized = zq_out.reshape(b, h, w, c).transpose(0, 3, 1, 2)
    loss = loss.reshape(b, h, w, c)
    com = com.reshape(b, h, w, c)
    cod = cod.reshape(b, h, w, c)
    return (z_quantized, loss, com, cod, indices)


# speedup vs baseline: 3.0766x; 1.0911x over previous
import functools

import jax
import jax.numpy as jnp
from jax import lax
from jax.experimental import pallas as pl
from jax.experimental.pallas import tpu as pltpu
from jax.experimental.pallas import tpu_sc as plsc

CB = 8192
C = 256
COMMIT = 0.25
N = 16384
ROW_BLK = 1024
N_STEPS = N // ROW_BLK


def _argmin_body(z_ref, e_ref, idx_ref, en_ref):
    z = z_ref[...]
    e = e_ref[...]
    zn = jnp.sum(z * z, axis=1, keepdims=True)

    @pl.when(pl.program_id(0) == 0)
    def _():
        en_ref[...] = jnp.sum(e * e, axis=1, keepdims=True)

    en = en_ref[...][:, 0]
    mm = lax.dot_general(z, e, (((1,), (1,)), ((), ())),
                         preferred_element_type=jnp.float32)
    d = (zn + en) - 2.0 * mm
    dmin = jnp.min(d, axis=1, keepdims=True)
    iota = lax.broadcasted_iota(jnp.int32, d.shape, 1)
    idx = jnp.min(jnp.where(d == dmin, iota, CB), axis=1)
    idx_ref[0, 0, :] = idx.astype(jnp.int32)


def _losses_body(z_ref, zq_ref, out_ref, loss_ref, com_ref, cod_ref):
    z = z_ref[...]
    zq = zq_ref[...]
    diff = zq - z
    d2 = diff * diff
    com = COMMIT * d2
    out_ref[...] = z + diff
    loss_ref[...] = com + d2
    com_ref[...] = com
    cod_ref[...] = d2


def _make_sc_gather():
    info = plsc.get_sparse_core_info()
    nw = info.num_cores * info.num_subcores
    b_per_w = N // nw
    chunk = 32
    nch = b_per_w // chunk
    mesh = plsc.VectorSubcoreMesh(core_axis_name="c", subcore_axis_name="s")

    @functools.partial(
        pl.kernel, mesh=mesh,
        out_type=jax.ShapeDtypeStruct((N, C), jnp.float32),
        scratch_types=[
            pltpu.VMEM((chunk,), jnp.int32),
            pltpu.VMEM((chunk, C), jnp.float32),
            pltpu.SemaphoreType.DMA,
        ],
    )
    def sc_gather(table_hbm, idx_hbm, out_hbm, idx_v, rows_v, sem):
        wid = lax.axis_index("s") * info.num_cores + lax.axis_index("c")
        base = wid * b_per_w

        def body(i, _):
            off = base + i * chunk
            pltpu.sync_copy(idx_hbm.at[pl.ds(off, chunk)], idx_v)
            pltpu.async_copy(table_hbm.at[idx_v], rows_v, sem).wait()
            pltpu.sync_copy(rows_v, out_hbm.at[pl.ds(off, chunk)])
            return ()

        lax.fori_loop(0, nch, body, (), unroll=False)

    return sc_gather


def kernel(z, embedding):
    b, c, h, w = z.shape
    z_flat = jnp.transpose(z.astype(jnp.float32), (0, 2, 3, 1)).reshape(-1, c)

    idx3 = pl.pallas_call(
        _argmin_body,
        grid=(N_STEPS,),
        in_specs=[
            pl.BlockSpec((ROW_BLK, c), lambda i: (i, 0)),
            pl.BlockSpec((CB, c), lambda i: (0, 0)),
        ],
        out_specs=pl.BlockSpec((1, 1, ROW_BLK), lambda i: (i, 0, 0)),
        out_shape=jax.ShapeDtypeStruct((N_STEPS, 1, ROW_BLK), jnp.int32),
        scratch_shapes=[pltpu.VMEM((CB, 1), jnp.float32)],
        compiler_params=pltpu.CompilerParams(
            dimension_semantics=("arbitrary",)),
    )(z_flat, embedding)
    indices = idx3.reshape(-1)

    zq = _make_sc_gather()(embedding, indices)

    LB = 2048
    zq_out, loss, com, cod = pl.pallas_call(
        _losses_body,
        grid=(N // LB,),
        in_specs=[
            pl.BlockSpec((LB, c), lambda i: (i, 0)),
            pl.BlockSpec((LB, c), lambda i: (i, 0)),
        ],
        out_specs=tuple(pl.BlockSpec((LB, c), lambda i: (i, 0)) for _ in range(4)),
        out_shape=tuple(jax.ShapeDtypeStruct((N, c), jnp.float32) for _ in range(4)),
        compiler_params=pltpu.CompilerParams(
            dimension_semantics=("arbitrary",)),
    )(z_flat, zq)

    z_quantized = zq_out.reshape(b, h, w, c).transpose(0, 3, 1, 2)
    loss = loss.reshape(b, h, w, c)
    com = com.reshape(b, h, w, c)
    cod = cod.reshape(b, h, w, c)
    return (z_quantized, loss, com, cod, indices)


# speedup vs baseline: 3.2208x; 1.0469x over previous
import functools

import jax
import jax.numpy as jnp
from jax import lax
from jax.experimental import pallas as pl
from jax.experimental.pallas import tpu as pltpu
from jax.experimental.pallas import tpu_sc as plsc

CB = 8192
C = 256
COMMIT = 0.25
N = 16384
ROW_BLK = 1024
N_STEPS = N // ROW_BLK


def _argmin_body(z_ref, e_ref, idx_ref, en_ref):
    z = z_ref[...]
    e = e_ref[...]
    zn = jnp.sum(z * z, axis=1, keepdims=True)

    @pl.when(pl.program_id(0) == 0)
    def _():
        en_ref[...] = jnp.sum(e * e, axis=1, keepdims=True)

    en = en_ref[...][:, 0]
    mm = lax.dot_general(z, e, (((1,), (1,)), ((), ())),
                         preferred_element_type=jnp.float32)
    d = (zn + en) - 2.0 * mm
    idx = jnp.argmin(d, axis=1)
    idx_ref[0, 0, :] = idx.astype(jnp.int32)


def _losses_body(z_ref, zq_ref, out_ref, loss_ref, com_ref, cod_ref):
    z = z_ref[...]
    zq = zq_ref[...]
    diff = zq - z
    d2 = diff * diff
    com = COMMIT * d2
    out_ref[...] = z + diff
    loss_ref[...] = com + d2
    com_ref[...] = com
    cod_ref[...] = d2


def _make_sc_gather():
    info = plsc.get_sparse_core_info()
    nw = info.num_cores * info.num_subcores
    b_per_w = N // nw
    chunk = 32
    nch = b_per_w // chunk
    mesh = plsc.VectorSubcoreMesh(core_axis_name="c", subcore_axis_name="s")

    @functools.partial(
        pl.kernel, mesh=mesh,
        out_type=jax.ShapeDtypeStruct((N, C), jnp.float32),
        scratch_types=[
            pltpu.VMEM((chunk,), jnp.int32),
            pltpu.VMEM((chunk, C), jnp.float32),
            pltpu.SemaphoreType.DMA,
        ],
    )
    def sc_gather(table_hbm, idx_hbm, out_hbm, idx_v, rows_v, sem):
        wid = lax.axis_index("s") * info.num_cores + lax.axis_index("c")
        base = wid * b_per_w

        def body(i, _):
            off = base + i * chunk
            pltpu.sync_copy(idx_hbm.at[pl.ds(off, chunk)], idx_v)
            pltpu.async_copy(table_hbm.at[idx_v], rows_v, sem).wait()
            pltpu.sync_copy(rows_v, out_hbm.at[pl.ds(off, chunk)])
            return ()

        lax.fori_loop(0, nch, body, (), unroll=False)

    return sc_gather


def kernel(z, embedding):
    b, c, h, w = z.shape
    z_flat = jnp.transpose(z.astype(jnp.float32), (0, 2, 3, 1)).reshape(-1, c)

    idx3 = pl.pallas_call(
        _argmin_body,
        grid=(N_STEPS,),
        in_specs=[
            pl.BlockSpec((ROW_BLK, c), lambda i: (i, 0)),
            pl.BlockSpec((CB, c), lambda i: (0, 0)),
        ],
        out_specs=pl.BlockSpec((1, 1, ROW_BLK), lambda i: (i, 0, 0)),
        out_shape=jax.ShapeDtypeStruct((N_STEPS, 1, ROW_BLK), jnp.int32),
        scratch_shapes=[pltpu.VMEM((CB, 1), jnp.float32)],
        compiler_params=pltpu.CompilerParams(
            dimension_semantics=("arbitrary",)),
    )(z_flat, embedding)
    indices = idx3.reshape(-1)

    zq = _make_sc_gather()(embedding, indices)

    LB = 2048
    zq_out, loss, com, cod = pl.pallas_call(
        _losses_body,
        grid=(N // LB,),
        in_specs=[
            pl.BlockSpec((LB, c), lambda i: (i, 0)),
            pl.BlockSpec((LB, c), lambda i: (i, 0)),
        ],
        out_specs=tuple(pl.BlockSpec((LB, c), lambda i: (i, 0)) for _ in range(4)),
        out_shape=tuple(jax.ShapeDtypeStruct((N, c), jnp.float32) for _ in range(4)),
        compiler_params=pltpu.CompilerParams(
            dimension_semantics=("arbitrary",)),
    )(z_flat, zq)

    z_quantized = zq_out.reshape(b, h, w, c).transpose(0, 3, 1, 2)
    loss = loss.reshape(b, h, w, c)
    com = com.reshape(b, h, w, c)
    cod = cod.reshape(b, h, w, c)
    return (z_quantized, loss, com, cod, indices)


# speedup vs baseline: 3.4169x; 1.0609x over previous
import functools

import jax
import jax.numpy as jnp
from jax import lax
from jax.experimental import pallas as pl
from jax.experimental.pallas import tpu as pltpu
from jax.experimental.pallas import tpu_sc as plsc

CB = 8192
C = 256
COMMIT = 0.25
N = 16384
HW = 1024
B = 16


def _argmin_body(z_ref, e_ref, idx_ref, en_ref):
    z = z_ref[0]
    e = e_ref[...]
    zn = jnp.sum(z * z, axis=0, keepdims=True)

    @pl.when(pl.program_id(0) == 0)
    def _():
        en_ref[...] = jnp.sum(e * e, axis=1, keepdims=True)

    en = en_ref[...]
    mm = lax.dot_general(e, z, (((1,), (0,)), ((), ())),
                         preferred_element_type=jnp.float32)
    d = (zn + en) - 2.0 * mm
    idx = jnp.argmin(d, axis=0)
    idx_ref[0, 0, :] = idx.astype(jnp.int32)


def _losses_body(z_ref, zq_ref, out_ref, loss_ref, com_ref, cod_ref):
    z = z_ref[0]
    zq = zq_ref[...]
    zqt = zq.T
    diff_t = zqt - z
    out_ref[0] = z + diff_t
    d2 = diff_t.T * diff_t.T
    com = COMMIT * d2
    loss_ref[...] = com + d2
    com_ref[...] = com
    cod_ref[...] = d2


def _make_sc_gather():
    info = plsc.get_sparse_core_info()
    nw = info.num_cores * info.num_subcores
    b_per_w = N // nw
    chunk = 32
    nch = b_per_w // chunk
    mesh = plsc.VectorSubcoreMesh(core_axis_name="c", subcore_axis_name="s")

    @functools.partial(
        pl.kernel, mesh=mesh,
        out_type=jax.ShapeDtypeStruct((N, C), jnp.float32),
        scratch_types=[
            pltpu.VMEM((chunk,), jnp.int32),
            pltpu.VMEM((chunk, C), jnp.float32),
            pltpu.SemaphoreType.DMA,
        ],
    )
    def sc_gather(table_hbm, idx_hbm, out_hbm, idx_v, rows_v, sem):
        wid = lax.axis_index("s") * info.num_cores + lax.axis_index("c")
        base = wid * b_per_w

        def body(i, _):
            off = base + i * chunk
            pltpu.sync_copy(idx_hbm.at[pl.ds(off, chunk)], idx_v)
            pltpu.async_copy(table_hbm.at[idx_v], rows_v, sem).wait()
            pltpu.sync_copy(rows_v, out_hbm.at[pl.ds(off, chunk)])
            return ()

        lax.fori_loop(0, nch, body, (), unroll=False)

    return sc_gather


def kernel(z, embedding):
    b, c, h, w = z.shape
    z3 = z.astype(jnp.float32).reshape(b, c, h * w)

    idx3 = pl.pallas_call(
        _argmin_body,
        grid=(B,),
        in_specs=[
            pl.BlockSpec((1, c, HW), lambda i: (i, 0, 0)),
            pl.BlockSpec((CB, c), lambda i: (0, 0)),
        ],
        out_specs=pl.BlockSpec((1, 1, HW), lambda i: (i, 0, 0)),
        out_shape=jax.ShapeDtypeStruct((B, 1, HW), jnp.int32),
        scratch_shapes=[pltpu.VMEM((CB, 1), jnp.float32)],
        compiler_params=pltpu.CompilerParams(
            dimension_semantics=("arbitrary",)),
    )(z3, embedding)
    indices = idx3.reshape(-1)

    zq = _make_sc_gather()(embedding, indices)

    zq_out, loss, com, cod = pl.pallas_call(
        _losses_body,
        grid=(B,),
        in_specs=[
            pl.BlockSpec((1, c, HW), lambda i: (i, 0, 0)),
            pl.BlockSpec((HW, c), lambda i: (i, 0)),
        ],
        out_specs=(
            pl.BlockSpec((1, c, HW), lambda i: (i, 0, 0)),
            pl.BlockSpec((HW, c), lambda i: (i, 0)),
            pl.BlockSpec((HW, c), lambda i: (i, 0)),
            pl.BlockSpec((HW, c), lambda i: (i, 0)),
        ),
        out_shape=(
            jax.ShapeDtypeStruct((B, c, HW), jnp.float32),
            jax.ShapeDtypeStruct((N, c), jnp.float32),
            jax.ShapeDtypeStruct((N, c), jnp.float32),
            jax.ShapeDtypeStruct((N, c), jnp.float32),
        ),
        compiler_params=pltpu.CompilerParams(
            dimension_semantics=("arbitrary",)),
    )(z3, zq)

    z_quantized = zq_out.reshape(b, c, h, w)
    loss = loss.reshape(b, h, w, c)
    com = com.reshape(b, h, w, c)
    cod = cod.reshape(b, h, w, c)
    return (z_quantized, loss, com, cod, indices)
